# baseline (speedup 1.0000x reference)
import jax
import jax.numpy as jnp
import numpy as np
from jax.experimental import pallas as pl

T = 32
F = 4
NB = 1000
ALPHA = 0.2
LOG2E = 1.4426950408889634


def _attention_stage(xin, fmat, hmat, nt):
    if nt:
        dn = (((1,), (1,)), ((), ()))
        Fm = jax.lax.dot_general(fmat, xin, dn,
                                 preferred_element_type=jnp.float32)
        hrep = jax.lax.dot_general(hmat, xin, dn,
                                   preferred_element_type=jnp.float32)
    else:
        Fm = jnp.dot(fmat, xin, preferred_element_type=jnp.float32)
        hrep = jnp.dot(hmat, xin, preferred_element_type=jnp.float32)
    f1 = Fm[:T, :]
    f2 = Fm[T:2 * T, :]
    f2a = Fm[2 * T:2 * T + 8 * T, :].reshape(T, 1, 8, NB)
    f2b = Fm[2 * T + 8 * T:, :].reshape(T, 1, 8, NB)
    mm = jnp.max(f1, axis=0, keepdims=True) + jnp.max(f2, axis=0, keepdims=True)
    mm = mm * LOG2E
    M = jnp.maximum(mm, ALPHA * mm)
    f1a = (f1 * LOG2E - M).reshape(1, F, 8, NB)
    f1b = (f1 * (ALPHA * LOG2E) - M).reshape(1, F, 8, NB)
    p = jnp.exp2(jnp.maximum(f2a + f1a, f2b + f1b))
    denom = jnp.sum(p, axis=0)
    inv = 1.0 / denom
    hb = hrep.reshape(F, T, 8, NB)
    outs = []
    for c in range(F):
        num = jnp.sum(p * hb[c][:, None, :, :], axis=0)
        o = num * inv
        o = jnp.where(o > 0, o, jnp.exp(jnp.minimum(o, 0.0)) - 1.0)
        outs.append(o.reshape(T, NB))
    return jnp.concatenate(outs, axis=0)


def _fused_kernel(x_ref, f1m_ref, h1m_ref, f2m_ref, h2m_ref, pm_ref, o_ref):
    y1 = _attention_stage(x_ref[:, :], f1m_ref[:, :], h1m_ref[:, :], True)
    y2 = _attention_stage(y1, f2m_ref[:, :], h2m_ref[:, :], False)
    o_ref[:, :] = jax.lax.dot_general(
        y2, pm_ref[:, :], (((0,), (1,)), ((), ())),
        preferred_element_type=jnp.float32)


def _build_mats(W1, a1, W2, a2):
    eye = jnp.eye(T, dtype=jnp.float32)
    mw1 = (W1.T[:, None, None, :] * eye[None, :, :, None]).reshape(F * T, T * F)
    mw2 = (W2.T[:, None, :, None] * eye[None, :, None, :]).reshape(F * T, F * T)

    def amat(a):
        top = (a[:F, 0][None, :, None] * eye[:, None, :]).reshape(T, F * T)
        bot = (a[F:, 0][None, :, None] * eye[:, None, :]).reshape(T, F * T)
        return jnp.concatenate([top, bot], axis=0)

    def stage_mats(mw, am):
        aw = jnp.dot(am, mw)
        f2w = aw[T:, :]
        fmat = jnp.concatenate([
            aw,
            jnp.repeat(f2w * LOG2E, 8, axis=0),
            jnp.repeat(f2w * (ALPHA * LOG2E), 8, axis=0),
        ], axis=0)
        hmat = jnp.repeat(mw, 8, axis=0)
        return fmat, hmat

    f1m, h1m = stage_mats(mw1, amat(a1))
    f2m, h2m = stage_mats(mw2, amat(a2))
    return f1m, h1m, f2m, h2m


def kernel(x, W1, a1, W2, a2):
    B, N, Tx, Fx = x.shape
    n_total = B * N
    assert n_total % NB == 0
    xt = x.reshape(n_total, Tx * Fx)

    f1m, h1m, f2m, h2m = _build_mats(W1, a1, W2, a2)
    perm = np.zeros((T * F, T * F), dtype=np.float32)
    tt = np.arange(T)
    for c in range(F):
        perm[tt * F + c, c * T + tt] = 1.0
    pmat = jnp.asarray(perm)

    grid = n_total // NB
    out = pl.pallas_call(
        _fused_kernel,
        grid=(grid,),
        in_specs=[
            pl.BlockSpec((NB, T * F), lambda i: (i, 0)),
            pl.BlockSpec((2 * T + 16 * T, T * F), lambda i: (0, 0)),
            pl.BlockSpec((8 * T * F, T * F), lambda i: (0, 0)),
            pl.BlockSpec((2 * T + 16 * T, T * F), lambda i: (0, 0)),
            pl.BlockSpec((8 * T * F, T * F), lambda i: (0, 0)),
            pl.BlockSpec((T * F, T * F), lambda i: (0, 0)),
        ],
        out_specs=pl.BlockSpec((NB, T * F), lambda i: (i, 0)),
        out_shape=jax.ShapeDtypeStruct((n_total, T * F), jnp.float32),
    )(xt, f1m, h1m, f2m, h2m, pmat)

    return out.reshape(B, N, Tx, Fx)


# speedup vs baseline: 1.0015x; 1.0015x over previous
import jax
import jax.numpy as jnp
import numpy as np
from jax.experimental import pallas as pl

T = 32
F = 4
NB = 1000
ALPHA = 0.2
LOG2E = 1.4426950408889634


def _attention_stage(xin, fmat, hmat, nt):
    if nt:
        dn = (((1,), (1,)), ((), ()))
        Fm = jax.lax.dot_general(fmat, xin, dn,
                                 preferred_element_type=jnp.float32)
        hrep = jax.lax.dot_general(hmat, xin, dn,
                                   preferred_element_type=jnp.float32)
    else:
        Fm = jnp.dot(fmat, xin, preferred_element_type=jnp.float32)
        hrep = jnp.dot(hmat, xin, preferred_element_type=jnp.float32)
    f1 = Fm[:T, :]
    f2 = Fm[T:2 * T, :]
    f2a = Fm[2 * T:2 * T + 8 * T, :].reshape(T, 1, 8, NB)
    f2b = Fm[2 * T + 8 * T:, :].reshape(T, 1, 8, NB)
    mm = jnp.max(f1, axis=0, keepdims=True) + jnp.max(f2, axis=0, keepdims=True)
    mm = mm * LOG2E
    M = jnp.maximum(mm, ALPHA * mm)
    f1a = (f1 * LOG2E - M).reshape(1, F, 8, NB)
    f1b = (f1 * (ALPHA * LOG2E) - M).reshape(1, F, 8, NB)
    p = jnp.exp2(jnp.maximum(f2a + f1a, f2b + f1b))
    denom = jnp.sum(p, axis=0)
    inv = 1.0 / denom
    hb = hrep.reshape(F, T, 8, NB)
    outs = []
    for c in range(F):
        num = jnp.sum(p * hb[c][:, None, :, :], axis=0)
        o = num * inv
        o = jnp.where(o > 0, o, jnp.exp(jnp.minimum(o, 0.0)) - 1.0)
        outs.append(o.reshape(T, NB))
    return jnp.concatenate(outs, axis=0)


def _fused_kernel(x_ref, f1m_ref, h1m_ref, f2m_ref, h2m_ref, pm_ref, o_ref):
    y1 = _attention_stage(x_ref[:, :], f1m_ref[:, :], h1m_ref[:, :], True)
    y2 = _attention_stage(y1, f2m_ref[:, :], h2m_ref[:, :], False)
    o_ref[:, :] = jax.lax.dot_general(
        y2, pm_ref[:, :], (((0,), (1,)), ((), ())),
        preferred_element_type=jnp.float32)


def _build_mats(W1, a1, W2, a2):
    eye = jnp.eye(T, dtype=jnp.float32)
    mw1 = (W1.T[:, None, :, None] * eye[None, :, None, :]).reshape(F * T, F * T)
    mw2 = (W2.T[:, None, :, None] * eye[None, :, None, :]).reshape(F * T, F * T)

    def amat(a):
        top = (a[:F, 0][None, :, None] * eye[:, None, :]).reshape(T, F * T)
        bot = (a[F:, 0][None, :, None] * eye[:, None, :]).reshape(T, F * T)
        return jnp.concatenate([top, bot], axis=0)

    def stage_mats(mw, am):
        aw = jnp.dot(am, mw)
        f2w = aw[T:, :]
        fmat = jnp.concatenate([
            aw,
            jnp.repeat(f2w * LOG2E, 8, axis=0),
            jnp.repeat(f2w * (ALPHA * LOG2E), 8, axis=0),
        ], axis=0)
        hmat = jnp.repeat(mw, 8, axis=0)
        return fmat, hmat

    f1m, h1m = stage_mats(mw1, amat(a1))
    f2m, h2m = stage_mats(mw2, amat(a2))
    return f1m, h1m, f2m, h2m


def kernel(x, W1, a1, W2, a2):
    B, N, Tx, Fx = x.shape
    n_total = B * N
    assert n_total % NB == 0
    xt = x.transpose(0, 1, 3, 2).reshape(n_total, Tx * Fx)

    f1m, h1m, f2m, h2m = _build_mats(W1, a1, W2, a2)
    pmat = jnp.eye(T * F, dtype=jnp.float32)

    grid = n_total // NB
    out = pl.pallas_call(
        _fused_kernel,
        grid=(grid,),
        in_specs=[
            pl.BlockSpec((NB, T * F), lambda i: (i, 0)),
            pl.BlockSpec((2 * T + 16 * T, T * F), lambda i: (0, 0)),
            pl.BlockSpec((8 * T * F, T * F), lambda i: (0, 0)),
            pl.BlockSpec((2 * T + 16 * T, T * F), lambda i: (0, 0)),
            pl.BlockSpec((8 * T * F, T * F), lambda i: (0, 0)),
            pl.BlockSpec((T * F, T * F), lambda i: (0, 0)),
        ],
        out_specs=pl.BlockSpec((NB, T * F), lambda i: (i, 0)),
        out_shape=jax.ShapeDtypeStruct((n_total, T * F), jnp.float32),
    )(xt, f1m, h1m, f2m, h2m, pmat)

    return out.reshape(B, N, Fx, Tx).transpose(0, 1, 3, 2)


# speedup vs baseline: 1.0105x; 1.0090x over previous
import jax
import jax.numpy as jnp
import numpy as np
from jax.experimental import pallas as pl

T = 32
F = 4
NB = 1000
ALPHA = 0.2
LOG2E = 1.4426950408889634


def _attention_stage(xin, fmat, hmat, nt):
    if nt:
        dn = (((1,), (1,)), ((), ()))
        Fm = jax.lax.dot_general(fmat, xin, dn,
                                 preferred_element_type=jnp.float32)
        hrep = jax.lax.dot_general(hmat, xin, dn,
                                   preferred_element_type=jnp.float32)
    else:
        Fm = jnp.dot(fmat, xin, preferred_element_type=jnp.float32)
        hrep = jnp.dot(hmat, xin, preferred_element_type=jnp.float32)
    f1 = Fm[:T, :]
    f2 = Fm[T:2 * T, :]
    f2a = Fm[2 * T:2 * T + 8 * T, :].reshape(T, 1, 8, NB)
    f2b = Fm[2 * T + 8 * T:, :].reshape(T, 1, 8, NB)
    mm = jnp.max(f1, axis=0, keepdims=True) + jnp.max(f2, axis=0, keepdims=True)
    mm = mm * LOG2E
    M = jnp.maximum(mm, ALPHA * mm)
    f1a = (f1 * LOG2E - M).reshape(1, F, 8, NB)
    f1b = (f1 * (ALPHA * LOG2E) - M).reshape(1, F, 8, NB)
    p = jnp.exp2(jnp.maximum(f2a + f1a, f2b + f1b))
    denom = jnp.sum(p, axis=0)
    inv = 1.0 / denom
    hb = hrep.reshape(F, T, 8, NB)
    outs = []
    for c in range(F):
        num = jnp.sum(p * hb[c][:, None, :, :], axis=0)
        o = num * inv
        o = jnp.where(o > 0, o, jnp.exp(jnp.minimum(o, 0.0)) - 1.0)
        outs.append(o.reshape(T, NB))
    return jnp.concatenate(outs, axis=0)


def _fused_kernel(x_ref, f1m_ref, h1m_ref, f2m_ref, h2m_ref, pm_ref, o_ref):
    y1 = _attention_stage(x_ref[:, :], f1m_ref[:, :], h1m_ref[:, :], True)
    y2 = _attention_stage(y1, f2m_ref[:, :], h2m_ref[:, :], False)
    o_ref[:, :] = jax.lax.dot_general(
        y2, pm_ref[:, :], (((0,), (1,)), ((), ())),
        preferred_element_type=jnp.float32)


def _build_mats(W1, a1, W2, a2):
    eye = jnp.eye(T, dtype=jnp.float32)
    mw1 = (W1.T[:, None, :, None] * eye[None, :, None, :]).reshape(F * T, F * T)
    mw2 = (W2.T[:, None, :, None] * eye[None, :, None, :]).reshape(F * T, F * T)

    def amat(a):
        top = (a[:F, 0][None, :, None] * eye[:, None, :]).reshape(T, F * T)
        bot = (a[F:, 0][None, :, None] * eye[:, None, :]).reshape(T, F * T)
        return jnp.concatenate([top, bot], axis=0)

    def stage_mats(mw, am):
        aw = jnp.dot(am, mw)
        f2w = aw[T:, :]
        fmat = jnp.concatenate([
            aw,
            jnp.repeat(f2w * LOG2E, 8, axis=0),
            jnp.repeat(f2w * (ALPHA * LOG2E), 8, axis=0),
        ], axis=0)
        hmat = jnp.repeat(mw, 8, axis=0)
        return fmat, hmat

    f1m, h1m = stage_mats(mw1, amat(a1))
    f2m, h2m = stage_mats(mw2, amat(a2))
    return f1m, h1m, f2m, h2m


def kernel(x, W1, a1, W2, a2):
    B, N, Tx, Fx = x.shape
    assert N % NB == 0
    f1m, h1m, f2m, h2m = _build_mats(W1, a1, W2, a2)
    pmat = jnp.eye(T * F, dtype=jnp.float32)

    call = pl.pallas_call(
        _fused_kernel,
        grid=(N // NB,),
        in_specs=[
            pl.BlockSpec((NB, T * F), lambda i: (i, 0)),
            pl.BlockSpec((2 * T + 16 * T, T * F), lambda i: (0, 0)),
            pl.BlockSpec((8 * T * F, T * F), lambda i: (0, 0)),
            pl.BlockSpec((2 * T + 16 * T, T * F), lambda i: (0, 0)),
            pl.BlockSpec((8 * T * F, T * F), lambda i: (0, 0)),
            pl.BlockSpec((T * F, T * F), lambda i: (0, 0)),
        ],
        out_specs=pl.BlockSpec((NB, T * F), lambda i: (i, 0)),
        out_shape=jax.ShapeDtypeStruct((N, T * F), jnp.float32),
    )

    outs = []
    for b in range(B):
        xb = x[b].transpose(0, 2, 1).reshape(N, Tx * Fx)
        ob = call(xb, f1m, h1m, f2m, h2m, pmat)
        outs.append(ob.reshape(N, Fx, Tx).transpose(0, 2, 1))
    return jnp.stack(outs, axis=0)


# speedup vs baseline: 1.0162x; 1.0057x over previous
import jax
import jax.numpy as jnp
import numpy as np
from jax.experimental import pallas as pl

T = 32
F = 4
NB = 2000
ALPHA = 0.2
LOG2E = 1.4426950408889634


def _attention_stage(xin, fmat, hmat, nt):
    if nt:
        dn = (((1,), (1,)), ((), ()))
        Fm = jax.lax.dot_general(fmat, xin, dn,
                                 preferred_element_type=jnp.float32)
        hrep = jax.lax.dot_general(hmat, xin, dn,
                                   preferred_element_type=jnp.float32)
    else:
        Fm = jnp.dot(fmat, xin, preferred_element_type=jnp.float32)
        hrep = jnp.dot(hmat, xin, preferred_element_type=jnp.float32)
    f1 = Fm[:T, :]
    f2 = Fm[T:2 * T, :]
    f2a = Fm[2 * T:2 * T + 8 * T, :].reshape(T, 1, 8, NB)
    f2b = Fm[2 * T + 8 * T:, :].reshape(T, 1, 8, NB)
    mm = jnp.max(f1, axis=0, keepdims=True) + jnp.max(f2, axis=0, keepdims=True)
    mm = mm * LOG2E
    M = jnp.maximum(mm, ALPHA * mm)
    f1a = (f1 * LOG2E - M).reshape(1, F, 8, NB)
    f1b = (f1 * (ALPHA * LOG2E) - M).reshape(1, F, 8, NB)
    p = jnp.exp2(jnp.maximum(f2a + f1a, f2b + f1b))
    denom = jnp.sum(p, axis=0)
    inv = 1.0 / denom
    hb = hrep.reshape(F, T, 8, NB)
    outs = []
    for c in range(F):
        num = jnp.sum(p * hb[c][:, None, :, :], axis=0)
        o = num * inv
        o = jnp.where(o > 0, o, jnp.exp(jnp.minimum(o, 0.0)) - 1.0)
        outs.append(o.reshape(T, NB))
    return jnp.concatenate(outs, axis=0)


def _fused_kernel(x_ref, f1m_ref, h1m_ref, f2m_ref, h2m_ref, pm_ref, o_ref):
    y1 = _attention_stage(x_ref[:, :], f1m_ref[:, :], h1m_ref[:, :], True)
    y2 = _attention_stage(y1, f2m_ref[:, :], h2m_ref[:, :], False)
    o_ref[:, :] = jax.lax.dot_general(
        y2, pm_ref[:, :], (((0,), (1,)), ((), ())),
        preferred_element_type=jnp.float32)


def _build_mats(W1, a1, W2, a2):
    eye = jnp.eye(T, dtype=jnp.float32)
    mw1 = (W1.T[:, None, :, None] * eye[None, :, None, :]).reshape(F * T, F * T)
    mw2 = (W2.T[:, None, :, None] * eye[None, :, None, :]).reshape(F * T, F * T)

    def amat(a):
        top = (a[:F, 0][None, :, None] * eye[:, None, :]).reshape(T, F * T)
        bot = (a[F:, 0][None, :, None] * eye[:, None, :]).reshape(T, F * T)
        return jnp.concatenate([top, bot], axis=0)

    def stage_mats(mw, am):
        aw = jnp.dot(am, mw)
        f2w = aw[T:, :]
        fmat = jnp.concatenate([
            aw,
            jnp.repeat(f2w * LOG2E, 8, axis=0),
            jnp.repeat(f2w * (ALPHA * LOG2E), 8, axis=0),
        ], axis=0)
        hmat = jnp.repeat(mw, 8, axis=0)
        return fmat, hmat

    f1m, h1m = stage_mats(mw1, amat(a1))
    f2m, h2m = stage_mats(mw2, amat(a2))
    return f1m, h1m, f2m, h2m


def kernel(x, W1, a1, W2, a2):
    B, N, Tx, Fx = x.shape
    n_total = B * N
    assert n_total % NB == 0
    f1m, h1m, f2m, h2m = _build_mats(W1, a1, W2, a2)
    pmat = jnp.eye(T * F, dtype=jnp.float32)

    call = pl.pallas_call(
        _fused_kernel,
        grid=(n_total // NB,),
        in_specs=[
            pl.BlockSpec((NB, T * F), lambda i: (i, 0)),
            pl.BlockSpec((2 * T + 16 * T, T * F), lambda i: (0, 0)),
            pl.BlockSpec((8 * T * F, T * F), lambda i: (0, 0)),
            pl.BlockSpec((2 * T + 16 * T, T * F), lambda i: (0, 0)),
            pl.BlockSpec((8 * T * F, T * F), lambda i: (0, 0)),
            pl.BlockSpec((T * F, T * F), lambda i: (0, 0)),
        ],
        out_specs=pl.BlockSpec((NB, T * F), lambda i: (i, 0)),
        out_shape=jax.ShapeDtypeStruct((n_total, T * F), jnp.float32),
    )

    xt = x.transpose(0, 1, 3, 2).reshape(n_total, Tx * Fx)
    out = call(xt, f1m, h1m, f2m, h2m, pmat)
    return out.reshape(B, N, Fx, Tx).transpose(0, 1, 3, 2)
